# Initial kernel scaffold; baseline (speedup 1.0000x reference)
import jax
import jax.numpy as jnp
from jax.experimental import pallas as pl


def kernel(hidden_states, router_w, router_b, gate_up_proj, gate_up_proj_bias, down_proj, down_proj_bias):
    raise NotImplementedError("write your pallas kernel here")


# speedup vs baseline: 1.6196x; 1.6196x over previous
import jax
import jax.numpy as jnp
from jax import lax
from jax.experimental import pallas as pl

E = 8
TOP_K = 2
H = 768
FF = 1536


def _moe_dense_kernel(x_ref, rw_ref, rb_ref, gw_ref, gb_ref, dw_ref, db_ref, out_ref):
    e = pl.program_id(0)
    t = pl.program_id(1)
    T = x_ref.shape[0]

    x = x_ref[...]

    logits = jnp.dot(x, rw_ref[...].T, preferred_element_type=jnp.float32) + rb_ref[...]
    idx = lax.broadcasted_iota(jnp.int32, logits.shape, 1)
    m1 = jnp.max(logits, axis=-1, keepdims=True)
    i1 = jnp.min(jnp.where(logits == m1, idx, E), axis=-1, keepdims=True)
    masked = jnp.where(idx == i1, -jnp.inf, logits)
    m2 = jnp.max(masked, axis=-1, keepdims=True)
    i2 = jnp.min(jnp.where(masked == m2, idx, E), axis=-1, keepdims=True)
    p1 = jax.nn.sigmoid(m1 - m2)
    p2 = 1.0 - p1
    w = jnp.where(i1 == e, p1, jnp.where(i2 == e, p2, 0.0))

    gu = jnp.dot(x, gw_ref[0], preferred_element_type=jnp.float32) + gb_ref[0]
    gate = gu[:, :FF]
    up = gu[:, FF:]
    gate = gate * jax.nn.sigmoid(1.702 * gate)
    act = (up + 1.0) * gate
    y = jnp.dot(act, dw_ref[0], preferred_element_type=jnp.float32) + db_ref[0]
    contrib = y * w

    rows = pl.ds(t * T, T)

    @pl.when(e == 0)
    def _():
        out_ref[rows, :] = contrib

    @pl.when(e != 0)
    def _():
        out_ref[rows, :] = out_ref[rows, :] + contrib


def kernel(hidden_states, router_w, router_b, gate_up_proj, gate_up_proj_bias,
           down_proj, down_proj_bias):
    b, s, h = hidden_states.shape
    n = b * s
    x = hidden_states.reshape(n, h)
    T = 512
    nt = n // T

    out = pl.pallas_call(
        _moe_dense_kernel,
        grid=(E, nt),
        in_specs=[
            pl.BlockSpec((T, h), lambda e, t: (t, 0)),
            pl.BlockSpec((E, h), lambda e, t: (0, 0)),
            pl.BlockSpec((1, E), lambda e, t: (0, 0)),
            pl.BlockSpec((1, h, 2 * FF), lambda e, t: (e, 0, 0)),
            pl.BlockSpec((1, 1, 2 * FF), lambda e, t: (e, 0, 0)),
            pl.BlockSpec((1, FF, h), lambda e, t: (e, 0, 0)),
            pl.BlockSpec((1, 1, h), lambda e, t: (e, 0, 0)),
        ],
        out_specs=pl.BlockSpec((n, h), lambda e, t: (0, 0)),
        out_shape=jax.ShapeDtypeStruct((n, h), jnp.float32),
    )(x, router_w, router_b.reshape(1, E), gate_up_proj,
      gate_up_proj_bias.reshape(E, 1, 2 * FF),
      down_proj, down_proj_bias.reshape(E, 1, h))
    return out.reshape(b, s, h)
